# Initial kernel scaffold; baseline (speedup 1.0000x reference)
import jax
import jax.numpy as jnp
from jax.experimental import pallas as pl


def kernel(x, edge_index, pred_connectivity, W1a, b1a, W1b, b1b, W2a, b2a, W2b, b2b, Wp, bp):
    raise NotImplementedError("write your pallas kernel here")


# speedup vs baseline: 5.6711x; 5.6711x over previous
import functools

import jax
import jax.numpy as jnp
from jax.experimental import pallas as pl

N = 19
E = 342
EP = 384
D_IN = 1025
HID = 256
OUT = 512


def _fused_kernel(dst_row_ref, src_col_ref, pc_col_ref, x_ref,
                  w1a_ref, b1a_ref, w1b_ref, b1b_ref,
                  w2a_ref, b2a_ref, w2b_ref, b2b_ref,
                  wp_ref, bp_ref, out_ref):
    f32 = jnp.float32
    dmask = (jax.lax.broadcasted_iota(jnp.int32, (N, EP), 0)
             == dst_row_ref[...]).astype(f32)
    smask = (jax.lax.broadcasted_iota(jnp.int32, (EP, N), 1)
             == src_col_ref[...]).astype(f32) * pc_col_ref[...]
    a = jnp.dot(dmask, smask, preferred_element_type=f32)
    eye = (jax.lax.broadcasted_iota(jnp.int32, (N, N), 0)
           == jax.lax.broadcasted_iota(jnp.int32, (N, N), 1)).astype(f32)
    apl = a + eye

    z1 = jnp.dot(apl, x_ref[...], preferred_element_type=f32)
    t1 = jax.nn.relu(jnp.dot(z1, w1a_ref[...], preferred_element_type=f32)
                     + b1a_ref[...])
    h1 = jax.nn.relu(jnp.dot(t1, w1b_ref[...], preferred_element_type=f32)
                     + b1b_ref[...])

    z2 = jnp.dot(apl, h1, preferred_element_type=f32)
    t2 = jax.nn.relu(jnp.dot(z2, w2a_ref[...], preferred_element_type=f32)
                     + b2a_ref[...])
    h2 = jnp.dot(t2, w2b_ref[...], preferred_element_type=f32) + b2b_ref[...]

    s = jnp.sum(h2 * wp_ref[...], axis=1, keepdims=True)
    total = jnp.sum(s, axis=0, keepdims=True) + bp_ref[...]
    out_ref[...] = jax.nn.sigmoid(total)


@functools.partial(jax.jit, static_argnames=("interpret",))
def _run(x, edge_index, pred_connectivity, W1a, b1a, W1b, b1b,
         W2a, b2a, W2b, b2b, Wp, bp, interpret=False):
    src = edge_index[0]
    dst = edge_index[1]
    pad = EP - E
    dst_row = jnp.pad(dst, (0, pad)).reshape(1, EP)
    src_col = jnp.pad(src, (0, pad)).reshape(EP, 1)
    pc_col = jnp.pad(pred_connectivity, (0, pad)).reshape(EP, 1)
    wp2d = Wp.reshape(N, OUT)
    out = pl.pallas_call(
        _fused_kernel,
        out_shape=jax.ShapeDtypeStruct((1, 1), jnp.float32),
        interpret=interpret,
    )(dst_row, src_col, pc_col, x,
      W1a, b1a.reshape(1, HID), W1b, b1b.reshape(1, HID),
      W2a, b2a.reshape(1, OUT), W2b, b2b.reshape(1, OUT),
      wp2d, bp.reshape(1, 1))
    return out.reshape(1)


def kernel(x, edge_index, pred_connectivity, W1a, b1a, W1b, b1b,
           W2a, b2a, W2b, b2b, Wp, bp):
    return _run(x, edge_index, pred_connectivity, W1a, b1a, W1b, b1b,
                W2a, b2a, W2b, b2b, Wp, bp)
